# Initial kernel scaffold; baseline (speedup 1.0000x reference)
import functools

import jax
import jax.numpy as jnp
from jax import lax
from jax.experimental import pallas as pl
from jax.experimental.pallas import tpu as pltpu
from jax.experimental.pallas import tpu_sc as plsc

_N = 200
_E = 12800
_NP = 256
_LANES = 16
_SUBC = 16
_ROWS = _NP // _SUBC
_BANK = _ROWS * _NP
_NSTEPS = _E // _LANES


def _sc_build_adj(ei1, ea1, ei2, ea2):
    mesh = plsc.VectorSubcoreMesh(core_axis_name="c", subcore_axis_name="s")

    @functools.partial(
        pl.kernel,
        out_type=jax.ShapeDtypeStruct((2, _SUBC, _BANK), jnp.float32),
        mesh=mesh,
        scratch_types=[
            pltpu.VMEM((_E,), jnp.int32),
            pltpu.VMEM((_E,), jnp.int32),
            pltpu.VMEM((_E,), jnp.float32),
            pltpu.VMEM((_LANES * _BANK,), jnp.float32),
            pltpu.VMEM((_BANK,), jnp.float32),
        ],
    )
    def build(ei1_h, ea1_h, ei2_h, ea2_h, out_h, src_v, dst_v, ew_v, bank_v,
              acc_v):
        c = lax.axis_index("c")
        s = lax.axis_index("s")
        base = s * _ROWS
        zeros = jnp.zeros((_LANES,), jnp.float32)
        lanes = lax.iota(jnp.int32, _LANES)

        def body(ei_h, ea_h, g):
            pltpu.sync_copy(ei_h.at[0], src_v)
            pltpu.sync_copy(ei_h.at[1], dst_v)
            pltpu.sync_copy(ea_h, ew_v)

            def zstep(i, carry):
                b0 = i * (_LANES * 8)
                for k in range(8):
                    bank_v[pl.ds(b0 + k * _LANES, _LANES)] = zeros
                return carry
            lax.fori_loop(0, (_LANES * _BANK) // (_LANES * 8), zstep, 0)

            def estep(i, carry):
                e0 = i * _LANES
                s16 = src_v[pl.ds(e0, _LANES)]
                d16 = dst_v[pl.ds(e0, _LANES)]
                w16 = ew_v[pl.ds(e0, _LANES)]
                w16 = jnp.where(s16 == d16, 0.0, w16)
                rel = d16 - base
                inr = (rel >= 0) & (rel < _ROWS)
                flat = jnp.where(inr, rel * _NP + s16, 0)
                idx = lanes * _BANK + flat
                plsc.addupdate_scatter(bank_v, [idx], w16, mask=inr)
                return carry
            lax.fori_loop(0, _NSTEPS, estep, 0)

            def rstep(j, carry):
                j0 = j * _LANES
                acc = zeros
                for l in range(_LANES):
                    acc = acc + bank_v[pl.ds(l * _BANK + j0, _LANES)]
                acc_v[pl.ds(j0, _LANES)] = acc
                return carry
            lax.fori_loop(0, _BANK // _LANES, rstep, 0)

            pltpu.sync_copy(acc_v, out_h.at[g, s])

        @pl.when(c == 0)
        def _():
            body(ei1_h, ea1_h, 0)

        @pl.when(c == 1)
        def _():
            body(ei2_h, ea2_h, 1)

    return build(ei1, ea1, ei2, ea2)


def _tc_forward(adj, x1p, x2p, gc1_W, gc1_b, gc4_W, gc4_b, cw1p, cb1p, cw2p,
                cb2p):
    def body(a_r, x1_r, x2_r, w1_r, b1_r, w4_r, b4_r, cw1_r, cb1_r, cw2_r,
             cb2_r, out_r):
        def make_l(A):
            deg = jnp.sum(A, axis=0)
            safe = jnp.where(deg > 0, deg, 1.0)
            dis = jnp.where(deg > 0, lax.rsqrt(safe), 0.0)
            return -(dis[:, None] * A * dis[None, :])

        def cheb(x, L, w_r, b):
            out = jnp.dot(x, w_r[0], preferred_element_type=jnp.float32)
            t1 = jnp.dot(L, x, preferred_element_type=jnp.float32)
            out = out + jnp.dot(t1, w_r[1], preferred_element_type=jnp.float32)
            t2 = 2.0 * jnp.dot(L, t1, preferred_element_type=jnp.float32) - x
            out = out + jnp.dot(t2, w_r[2], preferred_element_type=jnp.float32)
            return out + b

        def tower(x, L, b1, b4):
            h = jnp.maximum(cheb(x, L, w1_r, b1), 0.0)
            return jnp.maximum(cheb(h, L, w4_r, b4), 0.0)

        b1 = b1_r[...]
        b4 = b4_r[...]
        h1 = tower(x1_r[...], make_l(a_r[0]), b1, b4)
        h2 = tower(x2_r[...], make_l(a_r[1]), b1, b4)
        prod = h1 * h2
        hid = lax.dot_general(prod, cw1_r[...], (((0,), (0,)), ((), ())),
                              preferred_element_type=jnp.float32)
        hid = jnp.maximum(hid + cb1_r[...], 0.0)
        out_r[...] = jnp.dot(hid, cw2_r[...],
                             preferred_element_type=jnp.float32) + cb2_r[...]

    return pl.pallas_call(
        body,
        out_shape=jax.ShapeDtypeStruct((_NP, 128), jnp.float32),
    )(adj, x1p, x2p, gc1_W, gc1_b, gc4_W, gc4_b, cw1p, cb1p, cw2p, cb2p)


def kernel(x1, edge_index1, edge_attr1, x2, edge_index2, edge_attr2, gc1_W,
           gc1_b, gc4_W, gc4_b, cls_W1, cls_b1, cls_W2, cls_b2):
    adj = _sc_build_adj(edge_index1, edge_attr1, edge_index2, edge_attr2)
    adj = adj.reshape(2, _NP, _NP)

    pad_n = _NP - _N
    x1p = jnp.pad(x1, ((0, pad_n), (0, 0)))
    x2p = jnp.pad(x2, ((0, pad_n), (0, 0)))
    cw1p = jnp.pad(cls_W1, ((0, pad_n), (0, 28)))
    cb1p = jnp.pad(cls_b1, (0, 28)).reshape(1, 128)
    cw2p = jnp.pad(cls_W2, ((0, 28), (0, 127)))
    cb2p = jnp.pad(cls_b2, (0, 127)).reshape(1, 128)

    out = _tc_forward(adj, x1p, x2p, gc1_W, gc1_b.reshape(1, -1), gc4_W,
                      gc4_b.reshape(1, -1), cw1p, cb1p, cw2p, cb2p)
    return out[:, :1]


# speedup vs baseline: 24.6346x; 24.6346x over previous
import functools

import jax
import jax.numpy as jnp
from jax import lax
from jax.experimental import pallas as pl
from jax.experimental.pallas import tpu as pltpu
from jax.experimental.pallas import tpu_sc as plsc

_N = 200
_E = 12800
_NP = 256
_LANES = 16
_SUBC = 16
_ROWS = _NP // _SUBC
_BANK = _ROWS * _NP
_NSTEPS = _E // _LANES


def _sc_build_adj(ei1, ea1, ei2, ea2):
    mesh = plsc.VectorSubcoreMesh(core_axis_name="c", subcore_axis_name="s")

    @functools.partial(
        pl.kernel,
        out_type=jax.ShapeDtypeStruct((2, _SUBC, _BANK), jnp.float32),
        mesh=mesh,
        scratch_types=[
            pltpu.VMEM((_E,), jnp.int32),
            pltpu.VMEM((_E,), jnp.int32),
            pltpu.VMEM((_E,), jnp.float32),
            pltpu.VMEM((_LANES * _BANK,), jnp.float32),
            pltpu.VMEM((_BANK,), jnp.float32),
        ],
        compiler_params=pltpu.CompilerParams(needs_layout_passes=False),
    )
    def build(ei1_h, ea1_h, ei2_h, ea2_h, out_h, src_v, dst_v, ew_v, bank_v,
              acc_v):
        c = lax.axis_index("c")
        s = lax.axis_index("s")
        base = s * _ROWS
        zeros = jnp.zeros((_LANES,), jnp.float32)
        lanes = lax.iota(jnp.int32, _LANES)

        def body(ei_h, ea_h, g):
            pltpu.sync_copy(ei_h.at[0], src_v)
            pltpu.sync_copy(ei_h.at[1], dst_v)
            pltpu.sync_copy(ea_h, ew_v)

            def zstep(i, carry):
                b0 = i * (_LANES * 8)
                for k in range(8):
                    bank_v[pl.ds(b0 + k * _LANES, _LANES)] = zeros
                return carry
            lax.fori_loop(0, (_LANES * _BANK) // (_LANES * 8), zstep, 0)

            def estep(i, carry):
                e0 = i * _LANES
                s16 = src_v[pl.ds(e0, _LANES)]
                d16 = dst_v[pl.ds(e0, _LANES)]
                w16 = ew_v[pl.ds(e0, _LANES)]
                w16 = jnp.where(s16 == d16, 0.0, w16)
                rel = d16 - base
                inr = (rel >= 0) & (rel < _ROWS)
                flat = jnp.where(inr, rel * _NP + s16, 0)
                idx = lanes * _BANK + flat
                plsc.addupdate_scatter(bank_v, [idx], w16, mask=inr)
                return carry
            lax.fori_loop(0, _NSTEPS, estep, 0)

            def rstep(j, carry):
                j0 = j * _LANES
                acc = zeros
                for l in range(_LANES):
                    acc = acc + bank_v[pl.ds(l * _BANK + j0, _LANES)]
                acc_v[pl.ds(j0, _LANES)] = acc
                return carry
            lax.fori_loop(0, _BANK // _LANES, rstep, 0)

            pltpu.sync_copy(acc_v, out_h.at[g, s])

        @pl.when(c == 0)
        def _():
            body(ei1_h, ea1_h, 0)

        @pl.when(c == 1)
        def _():
            body(ei2_h, ea2_h, 1)

    return build(ei1, ea1, ei2, ea2)


def _tc_forward(adj, x1p, x2p, gc1_W, gc1_b, gc4_W, gc4_b, cw1p, cb1p, cw2p,
                cb2p):
    def body(a_r, x1_r, x2_r, w1_r, b1_r, w4_r, b4_r, cw1_r, cb1_r, cw2_r,
             cb2_r, out_r):
        def make_l(A):
            deg = jnp.sum(A, axis=0)
            safe = jnp.where(deg > 0, deg, 1.0)
            dis = jnp.where(deg > 0, lax.rsqrt(safe), 0.0)
            return -(dis[:, None] * A * dis[None, :])

        def cheb(x, L, w_r, b):
            out = jnp.dot(x, w_r[0], preferred_element_type=jnp.float32)
            t1 = jnp.dot(L, x, preferred_element_type=jnp.float32,
                         precision=lax.Precision.HIGHEST)
            out = out + jnp.dot(t1, w_r[1], preferred_element_type=jnp.float32)
            t2 = 2.0 * jnp.dot(L, t1, preferred_element_type=jnp.float32,
                               precision=lax.Precision.HIGHEST) - x
            out = out + jnp.dot(t2, w_r[2], preferred_element_type=jnp.float32)
            return out + b

        def tower(x, L, b1, b4):
            h = jnp.maximum(cheb(x, L, w1_r, b1), 0.0)
            return jnp.maximum(cheb(h, L, w4_r, b4), 0.0)

        b1 = b1_r[...]
        b4 = b4_r[...]
        h1 = tower(x1_r[...], make_l(a_r[0]), b1, b4)
        h2 = tower(x2_r[...], make_l(a_r[1]), b1, b4)
        prod = h1 * h2
        hid = lax.dot_general(prod, cw1_r[...], (((0,), (0,)), ((), ())),
                              preferred_element_type=jnp.float32)
        hid = jnp.maximum(hid + cb1_r[...], 0.0)
        out_r[...] = jnp.dot(hid, cw2_r[...],
                             preferred_element_type=jnp.float32) + cb2_r[...]

    return pl.pallas_call(
        body,
        out_shape=jax.ShapeDtypeStruct((_NP, 128), jnp.float32),
    )(adj, x1p, x2p, gc1_W, gc1_b, gc4_W, gc4_b, cw1p, cb1p, cw2p, cb2p)


def kernel(x1, edge_index1, edge_attr1, x2, edge_index2, edge_attr2, gc1_W,
           gc1_b, gc4_W, gc4_b, cls_W1, cls_b1, cls_W2, cls_b2):
    adj = _sc_build_adj(edge_index1, edge_attr1, edge_index2, edge_attr2)
    adj = adj.reshape(2, _NP, _NP)

    pad_n = _NP - _N
    x1p = jnp.pad(x1, ((0, pad_n), (0, 0)))
    x2p = jnp.pad(x2, ((0, pad_n), (0, 0)))
    cw1p = jnp.pad(cls_W1, ((0, pad_n), (0, 28)))
    cb1p = jnp.pad(cls_b1, (0, 28)).reshape(1, 128)
    cw2p = jnp.pad(cls_W2, ((0, 28), (0, 127)))
    cb2p = jnp.pad(cls_b2, (0, 127)).reshape(1, 128)

    out = _tc_forward(adj, x1p, x2p, gc1_W, gc1_b.reshape(1, -1), gc4_W,
                      gc4_b.reshape(1, -1), cw1p, cb1p, cw2p, cb2p)
    return out[:, :1]


# speedup vs baseline: 29.0484x; 1.1792x over previous
import functools

import jax
import jax.numpy as jnp
from jax import lax
from jax.experimental import pallas as pl
from jax.experimental.pallas import tpu as pltpu
from jax.experimental.pallas import tpu_sc as plsc

_N = 200
_E = 12800
_NP = 256
_LANES = 16
_SUBC = 16
_ROWS = _NP // _SUBC


def _sc_build_adj(ei1, ea1, ei2, ea2):
    mesh = plsc.VectorSubcoreMesh(core_axis_name="c", subcore_axis_name="s")

    @functools.partial(
        pl.kernel,
        out_type=jax.ShapeDtypeStruct((2, _NP, _NP), jnp.float32),
        mesh=mesh,
        scratch_types=[
            pltpu.VMEM((_E,), jnp.int32),
            pltpu.VMEM((_E,), jnp.int32),
            pltpu.VMEM((_E,), jnp.float32),
            pltpu.VMEM((_ROWS, _NP), jnp.float32),
        ],
        compiler_params=pltpu.CompilerParams(needs_layout_passes=False),
    )
    def build(ei1_h, ea1_h, ei2_h, ea2_h, out_h, src_v, dst_v, ew_v, acc_v):
        c = lax.axis_index("c")
        s = lax.axis_index("s")
        base = s * _ROWS
        zeros = jnp.zeros((_LANES,), jnp.float32)

        def body(ei_h, ea_h, g):
            pltpu.sync_copy(ei_h.at[0], src_v)
            pltpu.sync_copy(ei_h.at[1], dst_v)
            pltpu.sync_copy(ea_h, ew_v)

            def zstep(r, carry):
                for k in range(_NP // _LANES):
                    acc_v[r, pl.ds(k * _LANES, _LANES)] = zeros
                return carry
            lax.fori_loop(0, _ROWS, zstep, 0)

            def estep(i, carry):
                e0 = i * _LANES
                s16 = src_v[pl.ds(e0, _LANES)]
                d16 = dst_v[pl.ds(e0, _LANES)]
                w16 = ew_v[pl.ds(e0, _LANES)]
                w16 = jnp.where(s16 == d16, 0.0, w16)
                rel = d16 - base
                inr = (rel >= 0) & (rel < _ROWS)
                relc = jnp.where(inr, rel, 0)
                plsc.addupdate_scatter(acc_v, [relc, s16], w16, mask=inr)
                return carry
            lax.fori_loop(0, _E // _LANES, estep, 0)

            pltpu.sync_copy(acc_v, out_h.at[g, pl.ds(base, _ROWS)])

        @pl.when(c == 0)
        def _():
            body(ei1_h, ea1_h, 0)

        @pl.when(c == 1)
        def _():
            body(ei2_h, ea2_h, 1)

    return build(ei1, ea1, ei2, ea2)


def _tc_forward(adj, x1, x2, gc1_W, gc1_b, gc4_W, gc4_b, cls_W1, cls_b1,
                cls_W2, cls_b2):
    pad_n = _NP - _N

    def body(a_r, x1_r, x2_r, w1_r, b1_r, w4_r, b4_r, cw1_r, cb1_r, cw2_r,
             cb2_r, out_r):
        def make_l(A):
            deg = jnp.sum(A, axis=0)
            safe = jnp.where(deg > 0, deg, 1.0)
            dis = jnp.where(deg > 0, lax.rsqrt(safe), 0.0)
            return -(dis[:, None] * A * dis[None, :])

        def cheb(x, L, w_r, b):
            out = jnp.dot(x, w_r[0], preferred_element_type=jnp.float32)
            t1 = jnp.dot(L, x, preferred_element_type=jnp.float32,
                         precision=lax.Precision.HIGHEST)
            out = out + jnp.dot(t1, w_r[1], preferred_element_type=jnp.float32)
            t2 = 2.0 * jnp.dot(L, t1, preferred_element_type=jnp.float32,
                               precision=lax.Precision.HIGHEST) - x
            out = out + jnp.dot(t2, w_r[2], preferred_element_type=jnp.float32)
            return out + b

        def tower(x, L, b1, b4):
            h = jnp.maximum(cheb(x, L, w1_r, b1), 0.0)
            return jnp.maximum(cheb(h, L, w4_r, b4), 0.0)

        xpad = jnp.zeros((pad_n, x1_r.shape[1]), jnp.float32)
        x1p = jnp.concatenate([x1_r[...], xpad], axis=0)
        x2p = jnp.concatenate([x2_r[...], xpad], axis=0)
        b1 = b1_r[...]
        b4 = b4_r[...]
        h1 = tower(x1p, make_l(a_r[0]), b1, b4)
        h2 = tower(x2p, make_l(a_r[1]), b1, b4)
        prod = h1 * h2

        cw1 = jnp.pad(cw1_r[...], ((0, pad_n), (0, 28)))
        cb1 = jnp.pad(cb1_r[...], (0, 28))
        cw2 = jnp.pad(cw2_r[...], ((0, 28), (0, 0)))
        hid = lax.dot_general(prod, cw1, (((0,), (0,)), ((), ())),
                              preferred_element_type=jnp.float32)
        hid = jnp.maximum(hid + cb1, 0.0)
        out_r[...] = jnp.dot(hid, cw2,
                             preferred_element_type=jnp.float32) + cb2_r[...]

    return pl.pallas_call(
        body,
        out_shape=jax.ShapeDtypeStruct((_NP, 1), jnp.float32),
    )(adj, x1, x2, gc1_W, gc1_b, gc4_W, gc4_b, cls_W1, cls_b1, cls_W2, cls_b2)


def kernel(x1, edge_index1, edge_attr1, x2, edge_index2, edge_attr2, gc1_W,
           gc1_b, gc4_W, gc4_b, cls_W1, cls_b1, cls_W2, cls_b2):
    adj = _sc_build_adj(edge_index1, edge_attr1, edge_index2, edge_attr2)
    return _tc_forward(adj, x1, x2, gc1_W, gc1_b, gc4_W, gc4_b, cls_W1,
                       cls_b1, cls_W2, cls_b2)


# speedup vs baseline: 35.0611x; 1.2070x over previous
import functools

import jax
import jax.numpy as jnp
from jax import lax
from jax.experimental import pallas as pl
from jax.experimental.pallas import tpu as pltpu
from jax.experimental.pallas import tpu_sc as plsc

_N = 200
_E = 12800
_NP = 256
_LANES = 16
_SUBC = 16
_ROWS = _NP // _SUBC


_EPT = _E // _SUBC
_CH = 128
_NCH = (_EPT + _CH - 1) // _CH
_GRP = _CH // _LANES
_SLICE = _NP * _NP // _SUBC


def _sc_build_adj(ei1, ea1, ei2, ea2):
    mesh = plsc.VectorSubcoreMesh(core_axis_name="c", subcore_axis_name="s")

    @functools.partial(
        pl.kernel,
        out_type=jax.ShapeDtypeStruct((2, _NP, _NP), jnp.float32),
        mesh=mesh,
        scratch_types=[
            pltpu.VMEM((_EPT,), jnp.int32),
            pltpu.VMEM((_EPT,), jnp.int32),
            pltpu.VMEM((_EPT,), jnp.float32),
            pltpu.VMEM((_NCH, _CH), jnp.int32),
            pltpu.VMEM((_NCH, _CH), jnp.float32),
            pltpu.VMEM((_SLICE,), jnp.float32),
            pltpu.VMEM_SHARED((_NP * _NP,), jnp.float32),
            pltpu.SemaphoreType.DMA,
        ],
        compiler_params=pltpu.CompilerParams(needs_layout_passes=False),
    )
    def build(ei1_h, ea1_h, ei2_h, ea2_h, out_h, src_v, dst_v, ew_v, idx_v,
              val_v, zb_v, a_sh, sem):
        c = lax.axis_index("c")
        s = lax.axis_index("s")
        fzeros = jnp.zeros((_LANES,), jnp.float32)
        izeros = jnp.zeros((_LANES,), jnp.int32)

        def body(ei_h, ea_h, g):
            pltpu.sync_copy(ei_h.at[0, s], src_v)
            pltpu.sync_copy(ei_h.at[1, s], dst_v)
            pltpu.sync_copy(ea_h.at[s], ew_v)

            def zstep(i, carry):
                for k in range(8):
                    zb_v[pl.ds(i * (_LANES * 8) + k * _LANES, _LANES)] = fzeros
                return carry
            lax.fori_loop(0, _SLICE // (_LANES * 8), zstep, 0)
            pltpu.sync_copy(zb_v, a_sh.at[pl.ds(s * _SLICE, _SLICE)])

            for k in range(_GRP):
                idx_v[_NCH - 1, pl.ds(k * _LANES, _LANES)] = izeros
                val_v[_NCH - 1, pl.ds(k * _LANES, _LANES)] = fzeros

            def estep(i, carry):
                o = i * _LANES
                s16 = src_v[pl.ds(o, _LANES)]
                d16 = dst_v[pl.ds(o, _LANES)]
                w16 = ew_v[pl.ds(o, _LANES)]
                w16 = jnp.where(s16 == d16, 0.0, w16)
                row = i // _GRP
                col = (i % _GRP) * _LANES
                idx_v[row, pl.ds(col, _LANES)] = d16 * _NP + s16
                val_v[row, pl.ds(col, _LANES)] = w16
                return carry
            lax.fori_loop(0, _EPT // _LANES, estep, 0)

            plsc.subcore_barrier()

            hs = [pltpu.async_copy(val_v.at[j], a_sh.at[idx_v.at[j]], sem,
                                   add=True)
                  for j in range(_NCH)]
            for h in hs:
                h.wait()

            plsc.subcore_barrier()

            outs = [pltpu.async_copy(
                        a_sh.at[pl.ds((s * _ROWS + r) * _NP, _NP)],
                        out_h.at[g, s * _ROWS + r], sem)
                    for r in range(_ROWS)]
            for h in outs:
                h.wait()

        @pl.when(c == 0)
        def _():
            body(ei1_h, ea1_h, 0)

        @pl.when(c == 1)
        def _():
            body(ei2_h, ea2_h, 1)

    return build(ei1, ea1, ei2, ea2)


def _tc_forward(adj, x1, x2, gc1_W, gc1_b, gc4_W, gc4_b, cls_W1, cls_b1,
                cls_W2, cls_b2):
    pad_n = _NP - _N

    def body(a_r, x1_r, x2_r, w1_r, b1_r, w4_r, b4_r, cw1_r, cb1_r, cw2_r,
             cb2_r, out_r):
        def make_l(A):
            deg = jnp.sum(A, axis=0)
            safe = jnp.where(deg > 0, deg, 1.0)
            dis = jnp.where(deg > 0, lax.rsqrt(safe), 0.0)
            return -(dis[:, None] * A * dis[None, :])

        def cheb(x, L, w_r, b):
            out = jnp.dot(x, w_r[0], preferred_element_type=jnp.float32)
            t1 = jnp.dot(L, x, preferred_element_type=jnp.float32,
                         precision=lax.Precision.HIGHEST)
            out = out + jnp.dot(t1, w_r[1], preferred_element_type=jnp.float32)
            t2 = 2.0 * jnp.dot(L, t1, preferred_element_type=jnp.float32,
                               precision=lax.Precision.HIGHEST) - x
            out = out + jnp.dot(t2, w_r[2], preferred_element_type=jnp.float32)
            return out + b

        def tower(x, L, b1, b4):
            h = jnp.maximum(cheb(x, L, w1_r, b1), 0.0)
            return jnp.maximum(cheb(h, L, w4_r, b4), 0.0)

        xpad = jnp.zeros((pad_n, x1_r.shape[1]), jnp.float32)
        x1p = jnp.concatenate([x1_r[...], xpad], axis=0)
        x2p = jnp.concatenate([x2_r[...], xpad], axis=0)
        b1 = b1_r[...]
        b4 = b4_r[...]
        h1 = tower(x1p, make_l(a_r[0]), b1, b4)
        h2 = tower(x2p, make_l(a_r[1]), b1, b4)
        prod = h1 * h2

        cw1 = jnp.pad(cw1_r[...], ((0, pad_n), (0, 28)))
        cb1 = jnp.pad(cb1_r[...], (0, 28))
        cw2 = jnp.pad(cw2_r[...], ((0, 28), (0, 0)))
        hid = lax.dot_general(prod, cw1, (((0,), (0,)), ((), ())),
                              preferred_element_type=jnp.float32)
        hid = jnp.maximum(hid + cb1, 0.0)
        out_r[...] = jnp.dot(hid, cw2,
                             preferred_element_type=jnp.float32) + cb2_r[...]

    return pl.pallas_call(
        body,
        out_shape=jax.ShapeDtypeStruct((_NP, 1), jnp.float32),
    )(adj, x1, x2, gc1_W, gc1_b, gc4_W, gc4_b, cls_W1, cls_b1, cls_W2, cls_b2)


def kernel(x1, edge_index1, edge_attr1, x2, edge_index2, edge_attr2, gc1_W,
           gc1_b, gc4_W, gc4_b, cls_W1, cls_b1, cls_W2, cls_b2):
    adj = _sc_build_adj(edge_index1.reshape(2, _SUBC, _EPT),
                        edge_attr1.reshape(_SUBC, _EPT),
                        edge_index2.reshape(2, _SUBC, _EPT),
                        edge_attr2.reshape(_SUBC, _EPT))
    return _tc_forward(adj, x1, x2, gc1_W, gc1_b, gc4_W, gc4_b, cls_W1,
                       cls_b1, cls_W2, cls_b2)


# speedup vs baseline: 37.1111x; 1.0585x over previous
import functools

import jax
import jax.numpy as jnp
from jax import lax
from jax.experimental import pallas as pl
from jax.experimental.pallas import tpu as pltpu
from jax.experimental.pallas import tpu_sc as plsc

_N = 200
_E = 12800
_NP = 256
_LANES = 16
_SUBC = 16
_ROWS = _NP // _SUBC


_EPT = _E // _SUBC
_CH = 128
_NCH = (_EPT + _CH - 1) // _CH
_GRP = _CH // _LANES
_SLICE = _NP * _NP // _SUBC


def _sc_build_adj(ei1, ea1, ei2, ea2):
    mesh = plsc.VectorSubcoreMesh(core_axis_name="c", subcore_axis_name="s")

    @functools.partial(
        pl.kernel,
        out_type=jax.ShapeDtypeStruct((2, _NP, _NP), jnp.float32),
        mesh=mesh,
        scratch_types=[
            pltpu.VMEM((_EPT,), jnp.int32),
            pltpu.VMEM((_EPT,), jnp.int32),
            pltpu.VMEM((_EPT,), jnp.float32),
            pltpu.VMEM((_NCH, _CH), jnp.int32),
            pltpu.VMEM((_NCH, _CH), jnp.float32),
            pltpu.VMEM((_SLICE,), jnp.float32),
            pltpu.VMEM_SHARED((_NP * _NP,), jnp.float32),
            pltpu.SemaphoreType.DMA,
        ],
        compiler_params=pltpu.CompilerParams(needs_layout_passes=False),
    )
    def build(ei1_h, ea1_h, ei2_h, ea2_h, out_h, src_v, dst_v, ew_v, idx_v,
              val_v, zb_v, a_sh, sem):
        c = lax.axis_index("c")
        s = lax.axis_index("s")
        fzeros = jnp.zeros((_LANES,), jnp.float32)
        izeros = jnp.zeros((_LANES,), jnp.int32)

        def body(ei_h, ea_h, g):
            pltpu.sync_copy(ei_h.at[0, s], src_v)
            pltpu.sync_copy(ei_h.at[1, s], dst_v)
            pltpu.sync_copy(ea_h.at[s], ew_v)

            def zstep(i, carry):
                for k in range(8):
                    zb_v[pl.ds(i * (_LANES * 8) + k * _LANES, _LANES)] = fzeros
                return carry
            lax.fori_loop(0, _SLICE // (_LANES * 8), zstep, 0)
            pltpu.sync_copy(zb_v, a_sh.at[pl.ds(s * _SLICE, _SLICE)])

            for k in range(_GRP):
                idx_v[_NCH - 1, pl.ds(k * _LANES, _LANES)] = izeros
                val_v[_NCH - 1, pl.ds(k * _LANES, _LANES)] = fzeros

            def estep(i, carry):
                o = i * _LANES
                s16 = src_v[pl.ds(o, _LANES)]
                d16 = dst_v[pl.ds(o, _LANES)]
                w16 = ew_v[pl.ds(o, _LANES)]
                w16 = jnp.where(s16 == d16, 0.0, w16)
                row = i // _GRP
                col = (i % _GRP) * _LANES
                idx_v[row, pl.ds(col, _LANES)] = d16 * _NP + s16
                val_v[row, pl.ds(col, _LANES)] = w16
                return carry
            lax.fori_loop(0, _EPT // _LANES, estep, 0)

            plsc.subcore_barrier()

            hs = [pltpu.async_copy(val_v.at[j], a_sh.at[idx_v.at[j]], sem,
                                   add=True)
                  for j in range(_NCH)]
            for h in hs:
                h.wait()

            plsc.subcore_barrier()

            outs = [pltpu.async_copy(
                        a_sh.at[pl.ds((s * _ROWS + r) * _NP, _NP)],
                        out_h.at[g, s * _ROWS + r], sem)
                    for r in range(_ROWS)]
            for h in outs:
                h.wait()

        @pl.when(c == 0)
        def _():
            body(ei1_h, ea1_h, 0)

        @pl.when(c == 1)
        def _():
            body(ei2_h, ea2_h, 1)

    return build(ei1, ea1, ei2, ea2)


def _tc_forward(adj, x1, x2, gc1_W, gc1_b, gc4_W, gc4_b, cls_W1, cls_b1,
                cls_W2, cls_b2):
    pad_n = _NP - _N

    def body(a_r, x1_r, x2_r, w1_r, b1_r, w4_r, b4_r, cw1_r, cb1_r, cw2_r,
             cb2_r, out_r):
        def split(v):
            hi = v.astype(jnp.bfloat16)
            return hi, (v - hi.astype(jnp.float32)).astype(jnp.bfloat16)

        def make_l(A):
            deg = jnp.sum(A, axis=0)
            safe = jnp.where(deg > 0, deg, 1.0)
            dis = jnp.where(deg > 0, 1.0 / jnp.sqrt(safe), 0.0)
            return split(-(dis[:, None] * A * dis[None, :]))

        def prop(L, x):
            l_hi, l_lo = L
            x_hi, x_lo = split(x)
            t = jnp.dot(l_hi, x_hi, preferred_element_type=jnp.float32)
            t = t + jnp.dot(l_hi, x_lo, preferred_element_type=jnp.float32)
            return t + jnp.dot(l_lo, x_hi, preferred_element_type=jnp.float32)

        def cheb(x, L, w_r, b):
            out = jnp.dot(x, w_r[0], preferred_element_type=jnp.float32)
            t1 = prop(L, x)
            out = out + jnp.dot(t1, w_r[1], preferred_element_type=jnp.float32)
            t2 = 2.0 * prop(L, t1) - x
            out = out + jnp.dot(t2, w_r[2], preferred_element_type=jnp.float32)
            return out + b

        def tower(x, L, b1, b4):
            h = jnp.maximum(cheb(x, L, w1_r, b1), 0.0)
            return jnp.maximum(cheb(h, L, w4_r, b4), 0.0)

        xpad = jnp.zeros((pad_n, x1_r.shape[1]), jnp.float32)
        x1p = jnp.concatenate([x1_r[...], xpad], axis=0)
        x2p = jnp.concatenate([x2_r[...], xpad], axis=0)
        b1 = b1_r[...]
        b4 = b4_r[...]
        h1 = tower(x1p, make_l(a_r[0]), b1, b4)
        h2 = tower(x2p, make_l(a_r[1]), b1, b4)
        prod = h1 * h2

        cw1 = jnp.pad(cw1_r[...], ((0, pad_n), (0, 28)))
        cb1 = jnp.pad(cb1_r[...], (0, 28))
        cw2 = jnp.pad(cw2_r[...], ((0, 28), (0, 0)))
        hid = lax.dot_general(prod, cw1, (((0,), (0,)), ((), ())),
                              preferred_element_type=jnp.float32)
        hid = jnp.maximum(hid + cb1, 0.0)
        out_r[...] = jnp.dot(hid, cw2,
                             preferred_element_type=jnp.float32) + cb2_r[...]

    return pl.pallas_call(
        body,
        out_shape=jax.ShapeDtypeStruct((_NP, 1), jnp.float32),
    )(adj, x1, x2, gc1_W, gc1_b, gc4_W, gc4_b, cls_W1, cls_b1, cls_W2, cls_b2)


def kernel(x1, edge_index1, edge_attr1, x2, edge_index2, edge_attr2, gc1_W,
           gc1_b, gc4_W, gc4_b, cls_W1, cls_b1, cls_W2, cls_b2):
    adj = _sc_build_adj(edge_index1.reshape(2, _SUBC, _EPT),
                        edge_attr1.reshape(_SUBC, _EPT),
                        edge_index2.reshape(2, _SUBC, _EPT),
                        edge_attr2.reshape(_SUBC, _EPT))
    return _tc_forward(adj, x1, x2, gc1_W, gc1_b, gc4_W, gc4_b, cls_W1,
                       cls_b1, cls_W2, cls_b2)
